# Initial kernel scaffold; baseline (speedup 1.0000x reference)
import jax
import jax.numpy as jnp
from jax.experimental import pallas as pl


def kernel(coordinates, camera_center, far, hash_table):
    raise NotImplementedError("write your pallas kernel here")


# speedup vs baseline: 2.3251x; 2.3251x over previous
import functools

import jax
import jax.numpy as jnp
from jax import lax
from jax.experimental import pallas as pl
from jax.experimental.pallas import tpu as pltpu
from jax.experimental.pallas import tpu_sc as plsc

P2 = -1640531535
P3 = 805459861
HMASK = 2097151

NW = 32
CHUNK = 2048
GROUPS = CHUNK // 16


def _iota16():
    return lax.iota(jnp.int32, 16)


def _full16(v):
    return jnp.full((16,), v, jnp.int32)


def _hash_group(coords_v, lanes):
    c0 = plsc.load_gather(coords_v, [lanes, _full16(0)])
    c1 = plsc.load_gather(coords_v, [lanes, _full16(1)])
    c2 = plsc.load_gather(coords_v, [lanes, _full16(2)])
    return (c0 ^ (c1 * P2) ^ (c2 * P3)) & HMASK


def _rsqrt(s):
    i = plsc.bitcast(s, jnp.int32)
    i = 0x5F3759DF - (i >> 1)
    y = plsc.bitcast(i, jnp.float32)
    for _ in range(3):
        y = y * (1.5 - 0.5 * s * y * y)
    return y


def _sigmoid(x):
    return 1.0 / (1.0 + jnp.exp(-x))


def _stats_body(n_points, coords_hbm, table_hbm, out_hbm,
                coords_v, idx_v, rows_v, acc_v, sem):
    wid = lax.axis_index("c") * 16 + lax.axis_index("s")
    per_tile = n_points // NW
    nchunks = per_tile // CHUNK
    tile_base = wid * per_tile

    def chunk_body(ci, carry):
        s_acc, ss_acc = carry
        base = tile_base + ci * CHUNK
        pltpu.sync_copy(coords_hbm.at[pl.ds(base, CHUNK)], coords_v)

        def hash_body(j, _):
            lanes = j * 16 + _iota16()
            idx_v[pl.ds(j * 16, 16)] = _hash_group(coords_v, lanes)
            return 0

        lax.fori_loop(0, GROUPS, hash_body, 0)
        pltpu.async_copy(table_hbm.at[idx_v], rows_v, sem).wait()

        def acc_body(j, c):
            s, ss = c
            lanes = j * 16 + _iota16()
            for ch in range(3):
                v = plsc.load_gather(rows_v, [lanes, _full16(ch)])
                s = s + v
                ss = ss + v * v
            return (s, ss)

        return lax.fori_loop(0, GROUPS, acc_body, (s_acc, ss_acc))

    z = jnp.zeros((16,), jnp.float32)
    s_acc, ss_acc = lax.fori_loop(0, nchunks, chunk_body, (z, z))
    acc_v[pl.ds(0, 16)] = s_acc
    acc_v[pl.ds(16, 16)] = ss_acc
    pltpu.sync_copy(acc_v, out_hbm.at[wid])


def _main_body(n_points, coords_hbm, table_hbm, params_hbm, out_hbm,
               coords_v, idx_v, rows_v, out_v, params_v, sem):
    wid = lax.axis_index("c") * 16 + lax.axis_index("s")
    per_tile = n_points // NW
    nchunks = per_tile // CHUNK
    tile_base = wid * per_tile

    pltpu.sync_copy(params_hbm, params_v)
    pv = params_v[...]
    ones = jnp.ones((16,), jnp.float32)
    sA = pv[0] * ones
    c1 = pv[1] * ones
    off = [pv[2 + d] * ones for d in range(3)]

    def chunk_body(ci, _):
        base = tile_base + ci * CHUNK
        pltpu.sync_copy(coords_hbm.at[pl.ds(base, CHUNK)], coords_v)

        def hash_body(j, __):
            lanes = j * 16 + _iota16()
            idx_v[pl.ds(j * 16, 16)] = _hash_group(coords_v, lanes)
            return 0

        lax.fori_loop(0, GROUPS, hash_body, 0)
        pltpu.async_copy(table_hbm.at[idx_v], rows_v, sem).wait()

        def comp_body(j, __):
            lanes = j * 16 + _iota16()
            f = [plsc.load_gather(rows_v, [lanes, _full16(ch)])
                 for ch in range(14)]
            for d in range(3):
                cf = plsc.load_gather(coords_v, [lanes, _full16(d)]
                                      ).astype(jnp.float32)
                m = cf * sA + f[d] * c1 + off[d]
                plsc.store_scatter(out_v, [lanes, _full16(d)], m)
            r0, x0, y0, z0 = f[3], f[4], f[5], f[6]
            s = r0 * r0 + x0 * x0 + y0 * y0 + z0 * z0
            inv = 1.0 / (s * _rsqrt(s) + 1e-8)
            r, x, y, z = r0 * inv, x0 * inv, y0 * inv, z0 * inv
            sc = [_sigmoid(f[7 + d]) * sA for d in range(3)]
            R = [[1 - 2 * (y * y + z * z), 2 * (x * y - r * z), 2 * (x * z + r * y)],
                 [2 * (x * y + r * z), 1 - 2 * (x * x + z * z), 2 * (y * z - r * x)],
                 [2 * (x * z - r * y), 2 * (y * z + r * x), 1 - 2 * (x * x + y * y)]]
            L = [[R[i][jj] * sc[jj] for jj in range(3)] for i in range(3)]
            for i in range(3):
                for kk in range(i, 3):
                    cv = (L[i][0] * L[kk][0] + L[i][1] * L[kk][1]
                          + L[i][2] * L[kk][2])
                    plsc.store_scatter(out_v, [lanes, _full16(3 + i * 3 + kk)], cv)
                    if kk != i:
                        plsc.store_scatter(out_v, [lanes, _full16(3 + kk * 3 + i)], cv)
            for d in range(3):
                plsc.store_scatter(out_v, [lanes, _full16(12 + d)],
                                   _sigmoid(f[10 + d]))
            plsc.store_scatter(out_v, [lanes, _full16(15)],
                               _sigmoid(f[13] - 4.0))
            return 0

        lax.fori_loop(0, GROUPS, comp_body, 0)
        pltpu.sync_copy(out_v, out_hbm.at[pl.ds(base, CHUNK)])
        return 0

    lax.fori_loop(0, nchunks, chunk_body, 0)


def kernel(coordinates, camera_center, far, hash_table):
    n_points = coordinates.shape[0]
    nch, tsize = hash_table.shape
    table16 = jnp.zeros((tsize, 16), jnp.float32).at[:, :nch].set(hash_table.T)

    mesh = plsc.VectorSubcoreMesh(core_axis_name="c", subcore_axis_name="s")

    stats_call = pl.kernel(
        functools.partial(_stats_body, n_points),
        out_type=jax.ShapeDtypeStruct((NW, 32), jnp.float32),
        mesh=mesh,
        compiler_params=pltpu.CompilerParams(needs_layout_passes=False, use_tc_tiling_on_sc=False),
        scratch_types=[
            pltpu.VMEM((CHUNK, 3), jnp.int32),
            pltpu.VMEM((CHUNK,), jnp.int32),
            pltpu.VMEM((CHUNK, 16), jnp.float32),
            pltpu.VMEM((32,), jnp.float32),
            pltpu.SemaphoreType.DMA,
        ],
    )
    stats = stats_call(coordinates, table16)

    S = jnp.sum(stats[:, :16])
    SS = jnp.sum(stats[:, 16:])
    n = jnp.float32(3 * n_points)
    mu = S / n
    sigma = jnp.sqrt((SS - n * mu * mu) / (n - 1.0))

    far_s = far[0]
    sA = 2.0 * far_s / 128.0
    c1 = (sA / 6.0) / sigma
    off = camera_center - far_s + far_s / 128.0 - mu * c1
    params = (jnp.zeros((16,), jnp.float32)
              .at[0].set(sA).at[1].set(c1).at[2:5].set(off))

    main_call = pl.kernel(
        functools.partial(_main_body, n_points),
        out_type=jax.ShapeDtypeStruct((n_points, 16), jnp.float32),
        mesh=mesh,
        compiler_params=pltpu.CompilerParams(needs_layout_passes=False, use_tc_tiling_on_sc=False),
        scratch_types=[
            pltpu.VMEM((CHUNK, 3), jnp.int32),
            pltpu.VMEM((CHUNK,), jnp.int32),
            pltpu.VMEM((CHUNK, 16), jnp.float32),
            pltpu.VMEM((CHUNK, 16), jnp.float32),
            pltpu.VMEM((16,), jnp.float32),
            pltpu.SemaphoreType.DMA,
        ],
    )
    return main_call(coordinates, table16, params)


# speedup vs baseline: 2.8296x; 1.2170x over previous
import functools

import jax
import jax.numpy as jnp
from jax import lax
from jax.experimental import pallas as pl
from jax.experimental.pallas import tpu as pltpu
from jax.experimental.pallas import tpu_sc as plsc

P2 = -1640531535
P3 = 805459861
HMASK = 2097151

NW = 32
CHUNK = 2048
GROUPS = CHUNK // 16


def _iota16():
    return lax.iota(jnp.int32, 16)


def _full16(v):
    return jnp.full((16,), v, jnp.int32)


def _hash_group(coords_v, lanes):
    c0 = plsc.load_gather(coords_v, [lanes, _full16(0)])
    c1 = plsc.load_gather(coords_v, [lanes, _full16(1)])
    c2 = plsc.load_gather(coords_v, [lanes, _full16(2)])
    return (c0 ^ (c1 * P2) ^ (c2 * P3)) & HMASK


def _rsqrt(s):
    i = plsc.bitcast(s, jnp.int32)
    i = 0x5F3759DF - (i >> 1)
    y = plsc.bitcast(i, jnp.float32)
    for _ in range(3):
        y = y * (1.5 - 0.5 * s * y * y)
    return y


def _sigmoid(x):
    return 1.0 / (1.0 + jnp.exp(-x))


def _transpose_body(tsize, src_hbm, out_hbm, chan_v, out_v, sem):
    wid = lax.axis_index("c") * 16 + lax.axis_index("s")
    per_tile = tsize // NW
    nchunks = per_tile // CHUNK
    tile_base = wid * per_tile

    def chunk_body(ci, _):
        base = tile_base + ci * CHUNK
        cps = [pltpu.async_copy(src_hbm.at[c, pl.ds(base, CHUNK)],
                                chan_v.at[pl.ds(c * CHUNK, CHUNK)], sem)
               for c in range(14)]
        for cp in cps:
            cp.wait()

        def grp(j, __):
            lanes = j * 16 + _iota16()
            for c in range(14):
                v = chan_v[pl.ds(c * CHUNK + j * 16, 16)]
                plsc.store_scatter(out_v, [lanes, _full16(c)], v)
            return 0

        lax.fori_loop(0, GROUPS, grp, 0)
        pltpu.sync_copy(out_v, out_hbm.at[pl.ds(base, CHUNK)])
        return 0

    lax.fori_loop(0, nchunks, chunk_body, 0)


def _stats_body(n_points, coords_hbm, table_hbm, out_hbm,
                coords_v, idx_v, rows_v, acc_v, sem):
    wid = lax.axis_index("c") * 16 + lax.axis_index("s")
    per_tile = n_points // NW
    nchunks = per_tile // CHUNK
    tile_base = wid * per_tile

    def chunk_body(ci, carry):
        s_acc, ss_acc = carry
        base = tile_base + ci * CHUNK
        pltpu.sync_copy(coords_hbm.at[pl.ds(base, CHUNK)], coords_v)

        def hash_body(j, _):
            lanes = j * 16 + _iota16()
            idx_v[pl.ds(j * 16, 16)] = _hash_group(coords_v, lanes)
            return 0

        lax.fori_loop(0, GROUPS, hash_body, 0)
        pltpu.async_copy(table_hbm.at[idx_v], rows_v, sem).wait()

        def acc_body(j, c):
            s, ss = c
            lanes = j * 16 + _iota16()
            for ch in range(3):
                v = plsc.load_gather(rows_v, [lanes, _full16(ch)])
                s = s + v
                ss = ss + v * v
            return (s, ss)

        return lax.fori_loop(0, GROUPS, acc_body, (s_acc, ss_acc))

    z = jnp.zeros((16,), jnp.float32)
    s_acc, ss_acc = lax.fori_loop(0, nchunks, chunk_body, (z, z))
    acc_v[pl.ds(0, 16)] = s_acc
    acc_v[pl.ds(16, 16)] = ss_acc
    pltpu.sync_copy(acc_v, out_hbm.at[wid])


def _main_body(n_points, coords_hbm, table_hbm, params_hbm, out_hbm,
               coords_v, idx_v, rows_v, out_v, params_v, sem):
    wid = lax.axis_index("c") * 16 + lax.axis_index("s")
    per_tile = n_points // NW
    nchunks = per_tile // CHUNK
    tile_base = wid * per_tile

    pltpu.sync_copy(params_hbm, params_v)
    pv = params_v[...]
    ones = jnp.ones((16,), jnp.float32)
    sA = pv[0] * ones
    c1 = pv[1] * ones
    off = [pv[2 + d] * ones for d in range(3)]

    def chunk_body(ci, _):
        base = tile_base + ci * CHUNK
        pltpu.sync_copy(coords_hbm.at[pl.ds(base, CHUNK)], coords_v)

        def hash_body(j, __):
            lanes = j * 16 + _iota16()
            idx_v[pl.ds(j * 16, 16)] = _hash_group(coords_v, lanes)
            return 0

        lax.fori_loop(0, GROUPS, hash_body, 0)
        pltpu.async_copy(table_hbm.at[idx_v], rows_v, sem).wait()

        def comp_body(j, __):
            lanes = j * 16 + _iota16()
            f = [plsc.load_gather(rows_v, [lanes, _full16(ch)])
                 for ch in range(14)]
            for d in range(3):
                cf = plsc.load_gather(coords_v, [lanes, _full16(d)]
                                      ).astype(jnp.float32)
                m = cf * sA + f[d] * c1 + off[d]
                plsc.store_scatter(out_v, [lanes, _full16(d)], m)
            r0, x0, y0, z0 = f[3], f[4], f[5], f[6]
            s = r0 * r0 + x0 * x0 + y0 * y0 + z0 * z0
            inv = 1.0 / (s * _rsqrt(s) + 1e-8)
            r, x, y, z = r0 * inv, x0 * inv, y0 * inv, z0 * inv
            sc = [_sigmoid(f[7 + d]) * sA for d in range(3)]
            R = [[1 - 2 * (y * y + z * z), 2 * (x * y - r * z), 2 * (x * z + r * y)],
                 [2 * (x * y + r * z), 1 - 2 * (x * x + z * z), 2 * (y * z - r * x)],
                 [2 * (x * z - r * y), 2 * (y * z + r * x), 1 - 2 * (x * x + y * y)]]
            L = [[R[i][jj] * sc[jj] for jj in range(3)] for i in range(3)]
            for i in range(3):
                for kk in range(i, 3):
                    cv = (L[i][0] * L[kk][0] + L[i][1] * L[kk][1]
                          + L[i][2] * L[kk][2])
                    plsc.store_scatter(out_v, [lanes, _full16(3 + i * 3 + kk)], cv)
                    if kk != i:
                        plsc.store_scatter(out_v, [lanes, _full16(3 + kk * 3 + i)], cv)
            for d in range(3):
                plsc.store_scatter(out_v, [lanes, _full16(12 + d)],
                                   _sigmoid(f[10 + d]))
            plsc.store_scatter(out_v, [lanes, _full16(15)],
                               _sigmoid(f[13] - 4.0))
            return 0

        lax.fori_loop(0, GROUPS, comp_body, 0)
        pltpu.sync_copy(out_v, out_hbm.at[pl.ds(base, CHUNK)])
        return 0

    lax.fori_loop(0, nchunks, chunk_body, 0)


def kernel(coordinates, camera_center, far, hash_table):
    n_points = coordinates.shape[0]
    nch, tsize = hash_table.shape

    mesh = plsc.VectorSubcoreMesh(core_axis_name="c", subcore_axis_name="s")

    transpose_call = pl.kernel(
        functools.partial(_transpose_body, tsize),
        out_type=jax.ShapeDtypeStruct((tsize, 16), jnp.float32),
        mesh=mesh,
        compiler_params=pltpu.CompilerParams(needs_layout_passes=False, use_tc_tiling_on_sc=False),
        scratch_types=[
            pltpu.VMEM((14 * CHUNK,), jnp.float32),
            pltpu.VMEM((CHUNK, 16), jnp.float32),
            pltpu.SemaphoreType.DMA,
        ],
    )
    table16 = transpose_call(hash_table)

    stats_call = pl.kernel(
        functools.partial(_stats_body, n_points),
        out_type=jax.ShapeDtypeStruct((NW, 32), jnp.float32),
        mesh=mesh,
        compiler_params=pltpu.CompilerParams(needs_layout_passes=False, use_tc_tiling_on_sc=False),
        scratch_types=[
            pltpu.VMEM((CHUNK, 3), jnp.int32),
            pltpu.VMEM((CHUNK,), jnp.int32),
            pltpu.VMEM((CHUNK, 16), jnp.float32),
            pltpu.VMEM((32,), jnp.float32),
            pltpu.SemaphoreType.DMA,
        ],
    )
    stats = stats_call(coordinates, table16)

    S = jnp.sum(stats[:, :16])
    SS = jnp.sum(stats[:, 16:])
    n = jnp.float32(3 * n_points)
    mu = S / n
    sigma = jnp.sqrt((SS - n * mu * mu) / (n - 1.0))

    far_s = far[0]
    sA = 2.0 * far_s / 128.0
    c1 = (sA / 6.0) / sigma
    off = camera_center - far_s + far_s / 128.0 - mu * c1
    params = (jnp.zeros((16,), jnp.float32)
              .at[0].set(sA).at[1].set(c1).at[2:5].set(off))

    main_call = pl.kernel(
        functools.partial(_main_body, n_points),
        out_type=jax.ShapeDtypeStruct((n_points, 16), jnp.float32),
        mesh=mesh,
        compiler_params=pltpu.CompilerParams(needs_layout_passes=False, use_tc_tiling_on_sc=False),
        scratch_types=[
            pltpu.VMEM((CHUNK, 3), jnp.int32),
            pltpu.VMEM((CHUNK,), jnp.int32),
            pltpu.VMEM((CHUNK, 16), jnp.float32),
            pltpu.VMEM((CHUNK, 16), jnp.float32),
            pltpu.VMEM((16,), jnp.float32),
            pltpu.SemaphoreType.DMA,
        ],
    )
    return main_call(coordinates, table16, params)


# speedup vs baseline: 2.9493x; 1.0423x over previous
import functools

import jax
import jax.numpy as jnp
from jax import lax
from jax.experimental import pallas as pl
from jax.experimental.pallas import tpu as pltpu
from jax.experimental.pallas import tpu_sc as plsc

P2 = -1640531535
P3 = 805459861
HMASK = 2097151

NW = 32
CHUNK = 2048
GROUPS = CHUNK // 16


def _iota16():
    return lax.iota(jnp.int32, 16)


def _full16(v):
    return jnp.full((16,), v, jnp.int32)


def _hash_group(coords_v, lanes):
    c0 = plsc.load_gather(coords_v, [lanes, _full16(0)])
    c1 = plsc.load_gather(coords_v, [lanes, _full16(1)])
    c2 = plsc.load_gather(coords_v, [lanes, _full16(2)])
    return (c0 ^ (c1 * P2) ^ (c2 * P3)) & HMASK


def _rsqrt(s):
    i = plsc.bitcast(s, jnp.int32)
    i = 0x5F3759DF - (i >> 1)
    y = plsc.bitcast(i, jnp.float32)
    for _ in range(3):
        y = y * (1.5 - 0.5 * s * y * y)
    return y


def _sigmoid(x):
    return 1.0 / (1.0 + jnp.exp(-x))


def _transpose_body(tsize, src_hbm, out_hbm, chan_v, out_v, sem):
    wid = lax.axis_index("c") * 16 + lax.axis_index("s")
    per_tile = tsize // NW
    nchunks = per_tile // CHUNK
    tile_base = wid * per_tile

    def chunk_body(ci, _):
        base = pl.multiple_of(tile_base + ci * CHUNK, CHUNK)
        cps = [pltpu.async_copy(src_hbm.at[c, pl.ds(base, CHUNK)],
                                chan_v.at[pl.ds(c * CHUNK, CHUNK)], sem)
               for c in range(14)]
        for cp in cps:
            cp.wait()

        def grp(j, __):
            lanes = j * 16 + _iota16()
            m = lanes >> 3
            sub = (lanes & 7) << 4
            for c in range(14):
                v = chan_v[pl.ds(c * CHUNK + j * 16, 16)]
                plsc.store_scatter(out_v, [m, sub + c], v)
            return 0

        lax.fori_loop(0, GROUPS, grp, 0)
        base8 = pl.multiple_of(base // 8, CHUNK // 8)
        pltpu.sync_copy(out_v, out_hbm.at[pl.ds(base8, CHUNK // 8)])
        return 0

    lax.fori_loop(0, nchunks, chunk_body, 0)


def _stats_body(n_points, coords_hbm, table_hbm, out_hbm,
                coords_v, idx_v, rows_v, acc_v, sem):
    wid = lax.axis_index("c") * 16 + lax.axis_index("s")
    per_tile = n_points // NW
    nchunks = per_tile // CHUNK
    tile_base = wid * per_tile

    def chunk_body(ci, carry):
        s_acc, ss_acc = carry
        base = tile_base + ci * CHUNK
        pltpu.sync_copy(coords_hbm.at[pl.ds(base, CHUNK)], coords_v)

        def hash_body(j, _):
            lanes = j * 16 + _iota16()
            idx_v[pl.ds(j * 16, 16)] = _hash_group(coords_v, lanes)
            return 0

        lax.fori_loop(0, GROUPS, hash_body, 0)
        pltpu.async_copy(table_hbm.at[idx_v], rows_v, sem).wait()

        def acc_body(j, c):
            s, ss = c
            lanes = j * 16 + _iota16()
            for ch in range(3):
                v = plsc.load_gather(rows_v, [lanes, _full16(ch)])
                s = s + v
                ss = ss + v * v
            return (s, ss)

        return lax.fori_loop(0, GROUPS, acc_body, (s_acc, ss_acc))

    z = jnp.zeros((16,), jnp.float32)
    s_acc, ss_acc = lax.fori_loop(0, nchunks, chunk_body, (z, z))
    acc_v[pl.ds(0, 16)] = s_acc
    acc_v[pl.ds(16, 16)] = ss_acc
    pltpu.sync_copy(acc_v, out_hbm.at[wid])


def _main_body(n_points, coords_hbm, table_hbm, params_hbm, out_hbm,
               coords_v, idx_v, rows_v, out_v, params_v, sem):
    wid = lax.axis_index("c") * 16 + lax.axis_index("s")
    per_tile = n_points // NW
    nchunks = per_tile // CHUNK
    tile_base = wid * per_tile

    pltpu.sync_copy(params_hbm, params_v)
    pv = params_v[...]
    ones = jnp.ones((16,), jnp.float32)
    sA = pv[0] * ones
    c1 = pv[1] * ones
    off = [pv[2 + d] * ones for d in range(3)]

    def chunk_body(ci, _):
        base = tile_base + ci * CHUNK
        pltpu.sync_copy(coords_hbm.at[pl.ds(base, CHUNK)], coords_v)

        def hash_body(j, __):
            lanes = j * 16 + _iota16()
            idx_v[pl.ds(j * 16, 16)] = _hash_group(coords_v, lanes)
            return 0

        lax.fori_loop(0, GROUPS, hash_body, 0)
        pltpu.async_copy(table_hbm.at[idx_v], rows_v, sem).wait()

        def comp_body(j, __):
            lanes = j * 16 + _iota16()
            f = [plsc.load_gather(rows_v, [lanes, _full16(ch)])
                 for ch in range(14)]
            for d in range(3):
                cf = plsc.load_gather(coords_v, [lanes, _full16(d)]
                                      ).astype(jnp.float32)
                m = cf * sA + f[d] * c1 + off[d]
                plsc.store_scatter(out_v, [lanes, _full16(d)], m)
            r0, x0, y0, z0 = f[3], f[4], f[5], f[6]
            s = r0 * r0 + x0 * x0 + y0 * y0 + z0 * z0
            inv = 1.0 / (s * _rsqrt(s) + 1e-8)
            r, x, y, z = r0 * inv, x0 * inv, y0 * inv, z0 * inv
            sc = [_sigmoid(f[7 + d]) * sA for d in range(3)]
            R = [[1 - 2 * (y * y + z * z), 2 * (x * y - r * z), 2 * (x * z + r * y)],
                 [2 * (x * y + r * z), 1 - 2 * (x * x + z * z), 2 * (y * z - r * x)],
                 [2 * (x * z - r * y), 2 * (y * z + r * x), 1 - 2 * (x * x + y * y)]]
            L = [[R[i][jj] * sc[jj] for jj in range(3)] for i in range(3)]
            for i in range(3):
                for kk in range(i, 3):
                    cv = (L[i][0] * L[kk][0] + L[i][1] * L[kk][1]
                          + L[i][2] * L[kk][2])
                    plsc.store_scatter(out_v, [lanes, _full16(3 + i * 3 + kk)], cv)
                    if kk != i:
                        plsc.store_scatter(out_v, [lanes, _full16(3 + kk * 3 + i)], cv)
            for d in range(3):
                plsc.store_scatter(out_v, [lanes, _full16(12 + d)],
                                   _sigmoid(f[10 + d]))
            plsc.store_scatter(out_v, [lanes, _full16(15)],
                               _sigmoid(f[13] - 4.0))
            return 0

        lax.fori_loop(0, GROUPS, comp_body, 0)
        pltpu.sync_copy(out_v, out_hbm.at[pl.ds(base, CHUNK)])
        return 0

    lax.fori_loop(0, nchunks, chunk_body, 0)


def kernel(coordinates, camera_center, far, hash_table):
    n_points = coordinates.shape[0]
    nch, tsize = hash_table.shape

    mesh = plsc.VectorSubcoreMesh(core_axis_name="c", subcore_axis_name="s")

    transpose_call = pl.kernel(
        functools.partial(_transpose_body, tsize),
        out_type=jax.ShapeDtypeStruct((tsize // 8, 128), jnp.float32),
        mesh=mesh,
        compiler_params=pltpu.CompilerParams(needs_layout_passes=False, use_tc_tiling_on_sc=True),
        scratch_types=[
            pltpu.VMEM((14 * CHUNK,), jnp.float32),
            pltpu.VMEM((CHUNK // 8, 128), jnp.float32),
            pltpu.SemaphoreType.DMA,
        ],
    )
    table16 = jnp.reshape(transpose_call(hash_table), (tsize, 16))

    stats_call = pl.kernel(
        functools.partial(_stats_body, n_points),
        out_type=jax.ShapeDtypeStruct((NW, 32), jnp.float32),
        mesh=mesh,
        compiler_params=pltpu.CompilerParams(needs_layout_passes=False, use_tc_tiling_on_sc=False),
        scratch_types=[
            pltpu.VMEM((CHUNK, 3), jnp.int32),
            pltpu.VMEM((CHUNK,), jnp.int32),
            pltpu.VMEM((CHUNK, 16), jnp.float32),
            pltpu.VMEM((32,), jnp.float32),
            pltpu.SemaphoreType.DMA,
        ],
    )
    stats = stats_call(coordinates, table16)

    S = jnp.sum(stats[:, :16])
    SS = jnp.sum(stats[:, 16:])
    n = jnp.float32(3 * n_points)
    mu = S / n
    sigma = jnp.sqrt((SS - n * mu * mu) / (n - 1.0))

    far_s = far[0]
    sA = 2.0 * far_s / 128.0
    c1 = (sA / 6.0) / sigma
    off = camera_center - far_s + far_s / 128.0 - mu * c1
    params = (jnp.zeros((16,), jnp.float32)
              .at[0].set(sA).at[1].set(c1).at[2:5].set(off))

    main_call = pl.kernel(
        functools.partial(_main_body, n_points),
        out_type=jax.ShapeDtypeStruct((n_points, 16), jnp.float32),
        mesh=mesh,
        compiler_params=pltpu.CompilerParams(needs_layout_passes=False, use_tc_tiling_on_sc=False),
        scratch_types=[
            pltpu.VMEM((CHUNK, 3), jnp.int32),
            pltpu.VMEM((CHUNK,), jnp.int32),
            pltpu.VMEM((CHUNK, 16), jnp.float32),
            pltpu.VMEM((CHUNK, 16), jnp.float32),
            pltpu.VMEM((16,), jnp.float32),
            pltpu.SemaphoreType.DMA,
        ],
    )
    return main_call(coordinates, table16, params)


# speedup vs baseline: 7.4983x; 2.5424x over previous
import functools

import jax
import jax.numpy as jnp
from jax import lax
from jax.experimental import pallas as pl
from jax.experimental.pallas import tpu as pltpu
from jax.experimental.pallas import tpu_sc as plsc

P2 = -1640531535
P3 = 805459861
HMASK = 2097151

NW = 32
CHUNK = 2048
GROUPS = CHUNK // 16


def _iota16():
    return lax.iota(jnp.int32, 16)


def _full16(v):
    return jnp.full((16,), v, jnp.int32)


def _hash3(c0, c1, c2):
    return (c0 ^ (c1 * P2) ^ (c2 * P3)) & HMASK


def _rsqrt(s):
    i = plsc.bitcast(s, jnp.int32)
    i = 0x5F3759DF - (i >> 1)
    y = plsc.bitcast(i, jnp.float32)
    for _ in range(3):
        y = y * (1.5 - 0.5 * s * y * y)
    return y


def _sigmoid(x):
    return 1.0 / (1.0 + jnp.exp(-x))


def _transpose_body(tsize, src_hbm, out_hbm, chan_v, out_v, sem):
    wid = lax.axis_index("c") * 16 + lax.axis_index("s")
    per_tile = tsize // NW
    nchunks = per_tile // CHUNK
    tile_base = wid * per_tile

    def chunk_body(ci, _):
        base = pl.multiple_of(tile_base + ci * CHUNK, CHUNK)
        cps = [pltpu.async_copy(src_hbm.at[c, pl.ds(base, CHUNK)],
                                chan_v.at[pl.ds(c * CHUNK, CHUNK)], sem)
               for c in range(14)]
        for cp in cps:
            cp.wait()

        def grp(j, __):
            lanes = j * 16 + _iota16()
            m = lanes >> 3
            sub = (lanes & 7) << 4
            for c in range(14):
                v = chan_v[pl.ds(c * CHUNK + j * 16, 16)]
                plsc.store_scatter(out_v, [m, sub + c], v)
            return 0

        lax.fori_loop(0, GROUPS, grp, 0)
        base8 = pl.multiple_of(base // 8, CHUNK // 8)
        pltpu.sync_copy(out_v, out_hbm.at[pl.ds(base8, CHUNK // 8)])
        return 0

    lax.fori_loop(0, nchunks, chunk_body, 0)


def _stats_body(n_points, cx_hbm, cy_hbm, cz_hbm, table_hbm, out_hbm,
                coords_v, idx_v, rows_v, acc_v, sem):
    wid = lax.axis_index("c") * 16 + lax.axis_index("s")
    per_tile = n_points // NW
    nchunks = per_tile // CHUNK
    tile_base = wid * per_tile

    def chunk_body(ci, carry):
        s_acc, ss_acc = carry
        base = tile_base + ci * CHUNK
        for d, src in enumerate((cx_hbm, cy_hbm, cz_hbm)):
            pltpu.sync_copy(src.at[pl.ds(base, CHUNK)],
                            coords_v.at[pl.ds(d * CHUNK, CHUNK)])

        def hash_body(j, _):
            c0 = coords_v[pl.ds(j * 16, 16)]
            c1 = coords_v[pl.ds(CHUNK + j * 16, 16)]
            c2 = coords_v[pl.ds(2 * CHUNK + j * 16, 16)]
            idx_v[pl.ds(j * 16, 16)] = _hash3(c0, c1, c2)
            return 0

        lax.fori_loop(0, GROUPS, hash_body, 0)
        pltpu.async_copy(table_hbm.at[idx_v], rows_v, sem).wait()

        def acc_body(j, c):
            s, ss = c
            lanes = j * 16 + _iota16()
            for ch in range(3):
                v = plsc.load_gather(rows_v, [lanes, _full16(ch)])
                s = s + v
                ss = ss + v * v
            return (s, ss)

        return lax.fori_loop(0, GROUPS, acc_body, (s_acc, ss_acc))

    z = jnp.zeros((16,), jnp.float32)
    s_acc, ss_acc = lax.fori_loop(0, nchunks, chunk_body, (z, z))
    acc_v[pl.ds(0, 16)] = s_acc
    acc_v[pl.ds(16, 16)] = ss_acc
    pltpu.sync_copy(acc_v, out_hbm.at[wid])


def _main_body(n_points, cx_hbm, cy_hbm, cz_hbm, table_hbm, params_hbm, out_hbm,
               coords_v, idx_v, rows_v, out_v, params_v, sem):
    wid = lax.axis_index("c") * 16 + lax.axis_index("s")
    per_tile = n_points // NW
    nchunks = per_tile // CHUNK
    tile_base = wid * per_tile

    pltpu.sync_copy(params_hbm, params_v)
    pv = params_v[...]
    ones = jnp.ones((16,), jnp.float32)
    sA = pv[0] * ones
    c1 = pv[1] * ones
    off = [pv[2 + d] * ones for d in range(3)]

    def chunk_body(ci, _):
        base = tile_base + ci * CHUNK
        for d, src in enumerate((cx_hbm, cy_hbm, cz_hbm)):
            pltpu.sync_copy(src.at[pl.ds(base, CHUNK)],
                            coords_v.at[pl.ds(d * CHUNK, CHUNK)])

        def hash_body(j, __):
            c0 = coords_v[pl.ds(j * 16, 16)]
            c1_ = coords_v[pl.ds(CHUNK + j * 16, 16)]
            c2 = coords_v[pl.ds(2 * CHUNK + j * 16, 16)]
            idx_v[pl.ds(j * 16, 16)] = _hash3(c0, c1_, c2)
            return 0

        lax.fori_loop(0, GROUPS, hash_body, 0)
        pltpu.async_copy(table_hbm.at[idx_v], rows_v, sem).wait()

        def comp_body(j, __):
            lanes = j * 16 + _iota16()
            f = [plsc.load_gather(rows_v, [lanes, _full16(ch)])
                 for ch in range(14)]
            for d in range(3):
                cf = coords_v[pl.ds(d * CHUNK + j * 16, 16)].astype(jnp.float32)
                m = cf * sA + f[d] * c1 + off[d]
                plsc.store_scatter(out_v, [lanes, _full16(d)], m)
            r0, x0, y0, z0 = f[3], f[4], f[5], f[6]
            s = r0 * r0 + x0 * x0 + y0 * y0 + z0 * z0
            inv = 1.0 / (s * _rsqrt(s) + 1e-8)
            r, x, y, z = r0 * inv, x0 * inv, y0 * inv, z0 * inv
            sc = [_sigmoid(f[7 + d]) * sA for d in range(3)]
            R = [[1 - 2 * (y * y + z * z), 2 * (x * y - r * z), 2 * (x * z + r * y)],
                 [2 * (x * y + r * z), 1 - 2 * (x * x + z * z), 2 * (y * z - r * x)],
                 [2 * (x * z - r * y), 2 * (y * z + r * x), 1 - 2 * (x * x + y * y)]]
            L = [[R[i][jj] * sc[jj] for jj in range(3)] for i in range(3)]
            for i in range(3):
                for kk in range(i, 3):
                    cv = (L[i][0] * L[kk][0] + L[i][1] * L[kk][1]
                          + L[i][2] * L[kk][2])
                    plsc.store_scatter(out_v, [lanes, _full16(3 + i * 3 + kk)], cv)
                    if kk != i:
                        plsc.store_scatter(out_v, [lanes, _full16(3 + kk * 3 + i)], cv)
            for d in range(3):
                plsc.store_scatter(out_v, [lanes, _full16(12 + d)],
                                   _sigmoid(f[10 + d]))
            plsc.store_scatter(out_v, [lanes, _full16(15)],
                               _sigmoid(f[13] - 4.0))
            return 0

        lax.fori_loop(0, GROUPS, comp_body, 0)
        pltpu.sync_copy(out_v, out_hbm.at[pl.ds(base, CHUNK)])
        return 0

    lax.fori_loop(0, nchunks, chunk_body, 0)


def kernel(coordinates, camera_center, far, hash_table):
    n_points = coordinates.shape[0]
    nch, tsize = hash_table.shape

    mesh = plsc.VectorSubcoreMesh(core_axis_name="c", subcore_axis_name="s")

    transpose_call = pl.kernel(
        functools.partial(_transpose_body, tsize),
        out_type=jax.ShapeDtypeStruct((tsize // 8, 128), jnp.float32),
        mesh=mesh,
        compiler_params=pltpu.CompilerParams(needs_layout_passes=False, use_tc_tiling_on_sc=True),
        scratch_types=[
            pltpu.VMEM((14 * CHUNK,), jnp.float32),
            pltpu.VMEM((CHUNK // 8, 128), jnp.float32),
            pltpu.SemaphoreType.DMA,
        ],
    )
    table16 = jnp.reshape(transpose_call(hash_table), (tsize, 16))
    cx = coordinates[:, 0]
    cy = coordinates[:, 1]
    cz = coordinates[:, 2]

    stats_call = pl.kernel(
        functools.partial(_stats_body, n_points),
        out_type=jax.ShapeDtypeStruct((NW, 32), jnp.float32),
        mesh=mesh,
        compiler_params=pltpu.CompilerParams(needs_layout_passes=False, use_tc_tiling_on_sc=False),
        scratch_types=[
            pltpu.VMEM((3 * CHUNK,), jnp.int32),
            pltpu.VMEM((CHUNK,), jnp.int32),
            pltpu.VMEM((CHUNK, 16), jnp.float32),
            pltpu.VMEM((32,), jnp.float32),
            pltpu.SemaphoreType.DMA,
        ],
    )
    stats = stats_call(cx, cy, cz, table16)

    S = jnp.sum(stats[:, :16])
    SS = jnp.sum(stats[:, 16:])
    n = jnp.float32(3 * n_points)
    mu = S / n
    sigma = jnp.sqrt((SS - n * mu * mu) / (n - 1.0))

    far_s = far[0]
    sA = 2.0 * far_s / 128.0
    c1 = (sA / 6.0) / sigma
    off = camera_center - far_s + far_s / 128.0 - mu * c1
    params = (jnp.zeros((16,), jnp.float32)
              .at[0].set(sA).at[1].set(c1).at[2:5].set(off))

    main_call = pl.kernel(
        functools.partial(_main_body, n_points),
        out_type=jax.ShapeDtypeStruct((n_points, 16), jnp.float32),
        mesh=mesh,
        compiler_params=pltpu.CompilerParams(needs_layout_passes=False, use_tc_tiling_on_sc=False),
        scratch_types=[
            pltpu.VMEM((3 * CHUNK,), jnp.int32),
            pltpu.VMEM((CHUNK,), jnp.int32),
            pltpu.VMEM((CHUNK, 16), jnp.float32),
            pltpu.VMEM((CHUNK, 16), jnp.float32),
            pltpu.VMEM((16,), jnp.float32),
            pltpu.SemaphoreType.DMA,
        ],
    )
    return main_call(cx, cy, cz, table16, params)
